# baseline (speedup 1.0000x reference)
import functools

import jax
import jax.numpy as jnp
from jax import lax
from jax.experimental import pallas as pl
from jax.experimental.pallas import tpu as pltpu
from jax.experimental.pallas import tpu_sc as plsc

N_CAT_FIELDS = 26
N_NUM_FIELDS = 13
VOCAB = 100000
N_QUANTILES = 100
EMB_DIM = 32
N_CAT_PER_FIELD = VOCAB + 1
N_EMB_CAT = N_CAT_FIELDS * N_CAT_PER_FIELD
N_EMB_NUM = (N_QUANTILES + 2) * N_NUM_FIELDS
OUT_FIELDS = N_CAT_FIELDS + N_NUM_FIELDS

NC, NS, L = 2, 16, 16
NW = NC * NS
CHUNK = 128


def _sc_embed(xc_flat, xn_flat, mk_flat, cb16, nb16, cat_tab, num_tab, batch):
    cat_per_w = batch * N_CAT_FIELDS // NW
    num_per_w = batch * N_NUM_FIELDS // NW
    n_cat_chunks = cat_per_w // CHUNK
    n_num_chunks = num_per_w // CHUNK

    mesh = plsc.VectorSubcoreMesh(core_axis_name="c", subcore_axis_name="s")

    @functools.partial(
        pl.kernel,
        out_type=jax.ShapeDtypeStruct((batch * OUT_FIELDS, EMB_DIM), jnp.float32),
        mesh=mesh,
        scratch_types=[
            pltpu.VMEM((CHUNK,), jnp.int32),
            pltpu.VMEM((CHUNK,), jnp.int32),
            pltpu.VMEM((CHUNK,), jnp.int32),
            pltpu.VMEM((CHUNK, EMB_DIM), jnp.float32),
            pltpu.VMEM((CHUNK,), jnp.float32),
            pltpu.VMEM((CHUNK,), jnp.int32),
            pltpu.VMEM((CHUNK,), jnp.float32),
            pltpu.VMEM((CHUNK,), jnp.int32),
            pltpu.VMEM((CHUNK,), jnp.int32),
            pltpu.VMEM((CHUNK,), jnp.int32),
            pltpu.VMEM((CHUNK, EMB_DIM), jnp.float32),
            pltpu.VMEM((CHUNK, EMB_DIM), jnp.float32),
            pltpu.VMEM((CHUNK, EMB_DIM), jnp.float32),
            pltpu.VMEM((L,), jnp.int32),
            pltpu.VMEM((L,), jnp.int32),
            pltpu.SemaphoreType.DMA,
            pltpu.SemaphoreType.DMA,
        ],
        compiler_params=pltpu.CompilerParams(
            needs_layout_passes=False, use_tc_tiling_on_sc=False),
    )
    def body(xc_hbm, xn_hbm, mk_hbm, cb_hbm, nb_hbm, ctab_hbm, ntab_hbm,
             out_hbm, xc, cidx, coidx, crows, xn, mk, frac, bidxl, bidxh,
             noidx, blrows, bhrows, hbuf, cb_v, nb_v, sem, sem2):
        wid = lax.axis_index("s") * NC + lax.axis_index("c")
        pltpu.sync_copy(cb_hbm, cb_v)
        pltpu.sync_copy(nb_hbm, nb_v)
        cb = cb_v[...]
        nb = nb_v[...]
        lanes = lax.iota(jnp.int32, L)

        cat_row0 = wid * cat_per_w

        def cat_chunk(t, _):
            r0 = cat_row0 + t * CHUNK
            pltpu.sync_copy(xc_hbm.at[pl.ds(r0, CHUNK)], xc)
            for g in range(CHUNK // L):
                p = r0 + g * L + lanes
                fld = lax.rem(p, N_CAT_FIELDS)
                cidx[pl.ds(g * L, L)] = xc[pl.ds(g * L, L)] + fld * N_CAT_PER_FIELD + cb
                coidx[pl.ds(g * L, L)] = lax.div(p, N_CAT_FIELDS) * OUT_FIELDS + fld
            pltpu.async_copy(ctab_hbm.at[cidx], crows, sem).wait()
            pltpu.async_copy(crows, out_hbm.at[coidx], sem).wait()
            return _

        lax.fori_loop(0, n_cat_chunks, cat_chunk, 0)

        num_row0 = wid * num_per_w

        def num_chunk(t, _):
            r0 = num_row0 + t * CHUNK
            pltpu.sync_copy(xn_hbm.at[pl.ds(r0, CHUNK)], xn)
            pltpu.sync_copy(mk_hbm.at[pl.ds(r0, CHUNK)], mk)
            for g in range(CHUNK // L):
                x16 = xn[pl.ds(g * L, L)]
                x16 = jnp.minimum(jnp.maximum(x16, jnp.float32(1e-6)),
                                  jnp.float32(1.0 - 1e-6))
                xs = x16 * jnp.float32(N_QUANTILES)
                il = xs.astype(jnp.int32)
                frac[pl.ds(g * L, L)] = xs - il.astype(jnp.float32)
                m16 = mk[pl.ds(g * L, L)]
                q = r0 + g * L + lanes
                fld = lax.rem(q, N_NUM_FIELDS)
                rl = (il + 1) * m16 + fld * (N_QUANTILES + 2) + nb
                bidxl[pl.ds(g * L, L)] = rl
                bidxh[pl.ds(g * L, L)] = rl + m16
                noidx[pl.ds(g * L, L)] = (lax.div(q, N_NUM_FIELDS) * OUT_FIELDS
                                          + N_CAT_FIELDS + fld)
            cl = pltpu.async_copy(ntab_hbm.at[bidxl], blrows, sem)
            ch = pltpu.async_copy(ntab_hbm.at[bidxh], bhrows, sem2)
            cl.wait()
            ch.wait()
            for g in range(CHUNK // L):
                r16 = g * L + lanes
                fr = frac[pl.ds(g * L, L)]
                for d in range(EMB_DIM):
                    d16 = jnp.full((L,), d, jnp.int32)
                    vl = plsc.load_gather(blrows, [r16, d16])
                    vh = plsc.load_gather(bhrows, [r16, d16])
                    plsc.store_scatter(hbuf, [r16, d16], vl + fr * (vh - vl))
            pltpu.async_copy(hbuf, out_hbm.at[noidx], sem).wait()
            return _

        lax.fori_loop(0, n_num_chunks, num_chunk, 0)

    return body(xc_flat, xn_flat, mk_flat, cb16, nb16, cat_tab, num_tab)


def kernel(x_cat, x_num, mask, rand_table, emb_cat_table, emb_num_table):
    batch = x_cat.shape[0]
    rt = jnp.asarray(rand_table, jnp.int32)
    cb16 = jnp.full((L,), 1, jnp.int32) + rt * N_EMB_CAT
    nb16 = jnp.full((L,), 0, jnp.int32) + rt * N_EMB_NUM
    out = _sc_embed(
        x_cat.reshape(-1), x_num.reshape(-1), mask.reshape(-1),
        cb16, nb16, emb_cat_table, emb_num_table, batch)
    return out.reshape(batch, OUT_FIELDS, EMB_DIM)


# speedup vs baseline: 1.0193x; 1.0193x over previous
import functools

import jax
import jax.numpy as jnp
from jax import lax
from jax.experimental import pallas as pl
from jax.experimental.pallas import tpu as pltpu
from jax.experimental.pallas import tpu_sc as plsc

N_CAT_FIELDS = 26
N_NUM_FIELDS = 13
VOCAB = 100000
N_QUANTILES = 100
EMB_DIM = 32
N_CAT_PER_FIELD = VOCAB + 1
N_EMB_CAT = N_CAT_FIELDS * N_CAT_PER_FIELD
N_EMB_NUM = (N_QUANTILES + 2) * N_NUM_FIELDS
OUT_FIELDS = N_CAT_FIELDS + N_NUM_FIELDS

NC, NS, L = 2, 16, 16
NW = NC * NS
CHUNK = 128
CAT_SUPER = 13
NUM_SUPER = 4


def _sc_embed(xc_flat, xn_flat, mk_flat, cb16, nb16, cat_tab, num_tab, batch):
    cat_per_w = batch * N_CAT_FIELDS // NW
    num_per_w = batch * N_NUM_FIELDS // NW
    n_cat_chunks = cat_per_w // CHUNK
    n_num_chunks = num_per_w // CHUNK
    n_cat_supers = n_cat_chunks // CAT_SUPER

    mesh = plsc.VectorSubcoreMesh(core_axis_name="c", subcore_axis_name="s")

    @functools.partial(
        pl.kernel,
        out_type=jax.ShapeDtypeStruct((batch * OUT_FIELDS, EMB_DIM), jnp.float32),
        mesh=mesh,
        scratch_types=[
            pltpu.VMEM((cat_per_w,), jnp.int32),
            pltpu.VMEM((num_per_w,), jnp.float32),
            pltpu.VMEM((num_per_w,), jnp.int32),
            pltpu.VMEM((CAT_SUPER, CHUNK), jnp.int32),
            pltpu.VMEM((CAT_SUPER, CHUNK), jnp.int32),
            pltpu.VMEM((CAT_SUPER * CHUNK, EMB_DIM), jnp.float32),
            pltpu.VMEM((n_num_chunks, CHUNK), jnp.int32),
            pltpu.VMEM((n_num_chunks, CHUNK), jnp.int32),
            pltpu.VMEM((n_num_chunks, CHUNK), jnp.int32),
            pltpu.VMEM((num_per_w,), jnp.float32),
            pltpu.VMEM((NUM_SUPER * CHUNK, EMB_DIM), jnp.float32),
            pltpu.VMEM((NUM_SUPER * CHUNK, EMB_DIM), jnp.float32),
            pltpu.VMEM((NUM_SUPER * CHUNK, EMB_DIM), jnp.float32),
            pltpu.VMEM((L,), jnp.int32),
            pltpu.VMEM((L,), jnp.int32),
            pltpu.SemaphoreType.DMA,
            pltpu.SemaphoreType.DMA,
            pltpu.SemaphoreType.DMA,
            pltpu.SemaphoreType.DMA,
        ],
        compiler_params=pltpu.CompilerParams(
            needs_layout_passes=False, use_tc_tiling_on_sc=False),
    )
    def body(xc_hbm, xn_hbm, mk_hbm, cb_hbm, nb_hbm, ctab_hbm, ntab_hbm,
             out_hbm, xc_all, xn_all, mk_all, cidx, coidx, crows, nli, nhi,
             nout, frac, blrows, bhrows, hbuf, cb_v, nb_v,
             sem_cg, sem_cs, sem_ng, sem_ns):
        wid = lax.axis_index("s") * NC + lax.axis_index("c")
        pltpu.sync_copy(cb_hbm, cb_v)
        pltpu.sync_copy(nb_hbm, nb_v)
        cb = cb_v[...]
        nb = nb_v[...]
        lanes = lax.iota(jnp.int32, L)

        cat_row0 = wid * cat_per_w
        num_row0 = wid * num_per_w

        pltpu.sync_copy(xc_hbm.at[pl.ds(cat_row0, cat_per_w)], xc_all)
        pltpu.sync_copy(xn_hbm.at[pl.ds(num_row0, num_per_w)], xn_all)
        pltpu.sync_copy(mk_hbm.at[pl.ds(num_row0, num_per_w)], mk_all)

        def num_idx_chunk(t, _):
            for g in range(CHUNK // L):
                o = t * CHUNK + g * L
                x16 = xn_all[pl.ds(o, L)]
                x16 = jnp.minimum(jnp.maximum(x16, jnp.float32(1e-6)),
                                  jnp.float32(1.0 - 1e-6))
                xs = x16 * jnp.float32(N_QUANTILES)
                il = xs.astype(jnp.int32)
                frac[pl.ds(o, L)] = xs - il.astype(jnp.float32)
                m16 = mk_all[pl.ds(o, L)]
                q = num_row0 + o + lanes
                fld = lax.rem(q, N_NUM_FIELDS)
                rl = (il + 1) * m16 + fld * (N_QUANTILES + 2) + nb
                nli[t, pl.ds(g * L, L)] = rl
                nhi[t, pl.ds(g * L, L)] = rl + m16
                nout[t, pl.ds(g * L, L)] = (lax.div(q, N_NUM_FIELDS) * OUT_FIELDS
                                            + N_CAT_FIELDS + fld)
            return _

        lax.fori_loop(0, n_num_chunks, num_idx_chunk, 0)

        def cat_idx_chunk(sbase, t, _):
            for g in range(CHUNK // L):
                o = sbase + t * CHUNK + g * L
                p = cat_row0 + o + lanes
                fld = lax.rem(p, N_CAT_FIELDS)
                cidx[t, pl.ds(g * L, L)] = xc_all[pl.ds(o, L)] + fld * N_CAT_PER_FIELD + cb
                coidx[t, pl.ds(g * L, L)] = lax.div(p, N_CAT_FIELDS) * OUT_FIELDS + fld
            return _

        for s in range(n_cat_supers):
            lax.fori_loop(0, CAT_SUPER,
                          functools.partial(cat_idx_chunk, s * CAT_SUPER * CHUNK), 0)
            gds = [pltpu.async_copy(ctab_hbm.at[cidx.at[t]],
                                    crows.at[pl.ds(t * CHUNK, CHUNK)], sem_cg)
                   for t in range(CAT_SUPER)]
            for d in gds:
                d.wait()
            sds = [pltpu.async_copy(crows.at[pl.ds(t * CHUNK, CHUNK)],
                                    out_hbm.at[coidx.at[t]], sem_cs)
                   for t in range(CAT_SUPER)]
            for d in sds:
                d.wait()

        done = 0
        supers = []
        while done < n_num_chunks:
            nc = min(NUM_SUPER, n_num_chunks - done)
            supers.append((done, nc))
            done += nc

        for (c0, nc) in supers:
            gds = []
            for i in range(nc):
                gds.append(pltpu.async_copy(
                    ntab_hbm.at[nli.at[c0 + i]],
                    blrows.at[pl.ds(i * CHUNK, CHUNK)], sem_ng))
                gds.append(pltpu.async_copy(
                    ntab_hbm.at[nhi.at[c0 + i]],
                    bhrows.at[pl.ds(i * CHUNK, CHUNK)], sem_ng))
            for d in gds:
                d.wait()

            def interp_chunk(i, _):
                for g in range(CHUNK // L):
                    r16 = i * CHUNK + g * L + lanes
                    fr = frac[pl.ds(c0 * CHUNK + i * CHUNK + g * L, L)]
                    for dd in range(EMB_DIM):
                        d16 = jnp.full((L,), dd, jnp.int32)
                        vl = plsc.load_gather(blrows, [r16, d16])
                        vh = plsc.load_gather(bhrows, [r16, d16])
                        plsc.store_scatter(hbuf, [r16, d16], vl + fr * (vh - vl))
                return _

            lax.fori_loop(0, nc, interp_chunk, 0)
            sds = [pltpu.async_copy(hbuf.at[pl.ds(i * CHUNK, CHUNK)],
                                    out_hbm.at[nout.at[c0 + i]], sem_ns)
                   for i in range(nc)]
            for d in sds:
                d.wait()

    return body(xc_flat, xn_flat, mk_flat, cb16, nb16, cat_tab, num_tab)


def kernel(x_cat, x_num, mask, rand_table, emb_cat_table, emb_num_table):
    batch = x_cat.shape[0]
    rt = jnp.asarray(rand_table, jnp.int32)
    cb16 = jnp.full((L,), 1, jnp.int32) + rt * N_EMB_CAT
    nb16 = jnp.full((L,), 0, jnp.int32) + rt * N_EMB_NUM
    out = _sc_embed(
        x_cat.reshape(-1), x_num.reshape(-1), mask.reshape(-1),
        cb16, nb16, emb_cat_table, emb_num_table, batch)
    return out.reshape(batch, OUT_FIELDS, EMB_DIM)
